# baseline (speedup 1.0000x reference)
import functools

import jax
import jax.numpy as jnp
from jax import lax
from jax.experimental import pallas as pl
from jax.experimental.pallas import tpu as pltpu
from jax.experimental.pallas import tpu_sc as plsc

_NC, _NS = 2, 16
_NW = _NC * _NS
_L = 16
_H = 8
_GRP = 16


def _pick_ge(KPAD):
    for ge in (16, 8, 4, 2):
        if ge * KPAD <= 1024 and _GRP % ge == 0:
            return ge
    return 1


def _sc_sumk(m, ids, sph_p, bounds2d, Kmax):
    N, EMB = m.shape
    E, NSPH, KPAD = sph_p.shape
    EPW = E // _NW
    GE = _pick_ge(Kmax)
    GEKB = (GE * Kmax + 16 + 15) // 16 * 16
    NV = EMB // _L
    NACC = NSPH * NV
    ROWS = NSPH * EMB
    CNTB = GEKB // _L
    mesh = plsc.VectorSubcoreMesh(core_axis_name="c", subcore_axis_name="s")

    @functools.partial(
        pl.kernel,
        mesh=mesh,
        out_type=jax.ShapeDtypeStruct((ROWS, E), jnp.float32),
        compiler_params=pltpu.CompilerParams(
            use_tc_tiling_on_sc=False, needs_layout_passes=False),
        scratch_types=[
            pltpu.VMEM((2, GEKB, EMB), jnp.float32),
            pltpu.VMEM((2, GEKB), jnp.int32),
            pltpu.VMEM((2, GE, NSPH, KPAD), jnp.float32),
            pltpu.VMEM((2, ROWS, _GRP), jnp.float32),
            pltpu.VMEM((1, _L), jnp.int32),
            pltpu.SemaphoreType.DMA,
            pltpu.SemaphoreType.DMA,
            pltpu.SemaphoreType.DMA,
        ],
    )
    def k(m_hbm, ids_hbm, sph_hbm, bnd_hbm, out_hbm, m_v, ids_v, sph_v,
          stg_v, bnd_v, sem_in, sem_out0, sem_out1):
        sem_outs = (sem_out0, sem_out1)
        wid = lax.axis_index("s") * _NC + lax.axis_index("c")
        e_base = wid * EPW
        pltpu.sync_copy(bnd_hbm.at[pl.ds(wid, 1), :], bnd_v)
        ptr0 = bnd_v[0, :][0]
        lane = lax.broadcasted_iota(jnp.int32, (_L,), 0)
        zero = jnp.zeros((_L,), jnp.float32)
        SUBS = _GRP // GE
        NSG = (EPW // _GRP) * SUBS

        def dma_base(ptr):
            d = jnp.minimum(ptr - (ptr % 8), N - GEKB)
            return pl.multiple_of(d, 8)

        def fire(gsg, ptr_est, p):
            sub_e0 = e_base + gsg * GE
            d = dma_base(ptr_est)
            pltpu.async_copy(m_hbm.at[pl.ds(d, GEKB)], m_v.at[p], sem_in)
            pltpu.async_copy(ids_hbm.at[pl.ds(d, GEKB)], ids_v.at[p], sem_in)
            pltpu.async_copy(sph_hbm.at[pl.ds(sub_e0, GE)], sph_v.at[p],
                             sem_in)

        def wait_in(p):
            pltpu.make_async_copy(m_hbm.at[pl.ds(0, GEKB)], m_v.at[p],
                                  sem_in).wait()
            pltpu.make_async_copy(ids_hbm.at[pl.ds(0, GEKB)], ids_v.at[p],
                                  sem_in).wait()
            pltpu.make_async_copy(sph_hbm.at[pl.ds(0, GE)], sph_v.at[p],
                                  sem_in).wait()

        def scan_rows(p, sub_e0):
            def b_body(b, acc):
                idv = ids_v[p, pl.ds(b * _L, _L)]
                hit = jnp.logical_and(idv >= sub_e0, idv < sub_e0 + GE)
                return acc + plsc.all_reduce_population_count(hit)

            cnt = lax.fori_loop(0, GEKB // _L, b_body,
                                jnp.zeros((_L,), jnp.int32))
            return cnt[0]

        def edge_body(p, sp, e_loc, carry):
            ptr, sub_e0, dma_start = carry
            e = sub_e0 + e_loc
            start_local = ptr - dma_start
            base0 = start_local - (start_local % _L)
            base0 = pl.multiple_of(base0, _L)

            def cnt_cond(c):
                base, cnt = c
                may_continue = start_local + cnt >= base
                return jnp.logical_and(may_continue, base + _L <= GEKB)

            def cnt_body(c):
                base, cnt = c
                idv = ids_v[p, pl.ds(base, _L)]
                nm = plsc.all_reduce_population_count(idv == e)[0]
                return (base + _L, cnt + nm)

            _, seg_len = lax.while_loop(cnt_cond, cnt_body, (base0, 0))

            def kb_body(kb, accs):
                k0 = kb * _H
                wvs = []
                for s in range(NSPH):
                    wv = sph_v[p, e_loc, s, pl.ds(k0, _L)]
                    wvs.append(jnp.where(lane + k0 < seg_len, wv, 0.0))
                accs = list(accs)
                for t in range(_H):
                    local = start_local + k0 + t
                    rows = [m_v[p, local, pl.ds(j * _L, _L)]
                            for j in range(NV)]
                    for s in range(NSPH):
                        wsp = lax.broadcast_in_dim(wvs[s][t], (_L,), ())
                        for j in range(NV):
                            accs[s * NV + j] = accs[s * NV + j] + wsp * rows[j]
                return tuple(accs)

            nb = (seg_len + _H - 1) // _H
            accs = lax.fori_loop(0, nb, kb_body, (zero,) * NACC)

            e_col = jnp.broadcast_to((e - e_base) % _GRP, (_L,)).astype(
                jnp.int32)
            sp_idx = jnp.full((_L,), sp, jnp.int32)
            for s in range(NSPH):
                for j in range(NV):
                    idxr = lane + (s * EMB + j * _L)
                    plsc.store_scatter(stg_v, [sp_idx, idxr, e_col],
                                       accs[s * NV + j])
            return (ptr + seg_len, sub_e0, dma_start)

        def run_sub(gsg, ptr, p, sp):
            sub_e0 = e_base + gsg * GE
            dma_start = dma_base(ptr)
            wait_in(p)
            nxt_ptr = ptr + scan_rows(p, sub_e0)

            @pl.when(gsg + 1 < NSG)
            def _():
                fire(gsg + 1, nxt_ptr, 1 - p)

            body = functools.partial(edge_body, p, sp)
            ptr, _, _ = lax.fori_loop(0, GE, body, (ptr, sub_e0, dma_start))
            return ptr

        def pair_body(i, ptr):
            for half in range(2):
                g = 2 * i + half

                @pl.when(i >= 1)
                def _():
                    pltpu.make_async_copy(
                        stg_v.at[half],
                        out_hbm.at[:, pl.ds(0, _GRP)], sem_outs[half]).wait()

                for sub in range(SUBS):
                    j = half * SUBS + sub
                    ptr = run_sub(g * SUBS + sub, ptr, j % 2, half)
                col0 = e_base + g * _GRP
                pltpu.async_copy(stg_v.at[half],
                                 out_hbm.at[:, pl.ds(col0, _GRP)],
                                 sem_outs[half])
            return ptr

        fire(0, ptr0, 0)
        lax.fori_loop(0, EPW // _GRP // 2, pair_body, ptr0)
        for half in range(2):
            pltpu.make_async_copy(stg_v.at[half],
                                  out_hbm.at[:, pl.ds(0, _GRP)],
                                  sem_outs[half]).wait()

    return k(m, ids, sph_p, bounds2d)


def _tc_body(a_ref, st_ref, wf_ref, out_ref):
    INTERM, NSPH, BE = a_ref.shape
    EMB = st_ref.shape[0] // NSPH
    st = st_ref[...].reshape(NSPH, EMB, BE)
    a = a_ref[...]
    h = jnp.zeros((INTERM, EMB, BE), jnp.float32)
    for s in range(NSPH):
        h = h + a[:, s, :][:, None, :] * st[s][None, :, :]
    hf = h.reshape(INTERM * EMB, BE)
    out_ref[...] = lax.dot_general(
        hf, wf_ref[...], (((0,), (0,)), ((), ())),
        preferred_element_type=jnp.float32)


def _tc_compute(a_t, sumk_t, wf, BE=512):
    INTERM, NSPH, E = a_t.shape
    ROWS = sumk_t.shape[0]
    WK, UNITS = wf.shape
    grid = (E // BE,)
    return pl.pallas_call(
        _tc_body,
        grid=grid,
        in_specs=[
            pl.BlockSpec((INTERM, NSPH, BE), lambda i: (0, 0, i)),
            pl.BlockSpec((ROWS, BE), lambda i: (0, i)),
            pl.BlockSpec((WK, UNITS), lambda i: (0, 0)),
        ],
        out_specs=pl.BlockSpec((BE, UNITS), lambda i: (i, 0)),
        out_shape=jax.ShapeDtypeStruct((E, UNITS), jnp.float32),
        compiler_params=pltpu.CompilerParams(
            dimension_semantics=("arbitrary",)),
    )(a_t, sumk_t, wf)


def kernel(rbf_W1, sph, m, weight, id_reduce, id_ragged_idx):
    E, INTERM, NSPH = rbf_W1.shape
    Kmax = sph.shape[2]
    N, EMB = m.shape
    UNITS = weight.shape[2]

    ids = id_reduce.astype(jnp.int32)
    EPW = E // _NW
    qs = jnp.arange(_NW + 1, dtype=jnp.int32) * EPW
    bounds = jnp.searchsorted(ids, qs, side="left").astype(jnp.int32)
    bounds = jnp.concatenate([bounds, jnp.full((15,), N, jnp.int32)])
    bounds2d = jnp.tile(bounds[:, None], (1, _L))

    KPAD = (Kmax + _H - 1) // _H * _H + _H
    sph_p = jnp.pad(sph, ((0, 0), (0, 0), (0, KPAD - Kmax)))

    sumk_t = _sc_sumk(m, ids, sph_p, bounds2d, Kmax)
    a_t = jnp.transpose(rbf_W1, (1, 2, 0))
    wf = jnp.transpose(weight, (1, 0, 2)).reshape(INTERM * EMB, UNITS)
    return _tc_compute(a_t, sumk_t, wf)


# speedup vs baseline: 1.0544x; 1.0544x over previous
import functools

import jax
import jax.numpy as jnp
from jax import lax
from jax.experimental import pallas as pl
from jax.experimental.pallas import tpu as pltpu
from jax.experimental.pallas import tpu_sc as plsc

_NC, _NS = 2, 16
_NW = _NC * _NS
_L = 16
_H = 16
_GRP = 16


def _pick_ge(KPAD):
    for ge in (16, 8, 4, 2):
        if ge * KPAD <= 1024 and _GRP % ge == 0:
            return ge
    return 1


def _sc_sumk(m, ids, sph_p, bounds2d, Kmax):
    N, EMB = m.shape
    E, NSPH, KPAD = sph_p.shape
    EPW = E // _NW
    GE = _pick_ge(Kmax)
    GEKB = (GE * Kmax + 8 + _H + 15) // 16 * 16
    NV = EMB // _L
    NACC = NSPH * NV
    ROWS = NSPH * EMB
    CNTB = GEKB // _L
    mesh = plsc.VectorSubcoreMesh(core_axis_name="c", subcore_axis_name="s")

    @functools.partial(
        pl.kernel,
        mesh=mesh,
        out_type=jax.ShapeDtypeStruct((ROWS, E), jnp.float32),
        compiler_params=pltpu.CompilerParams(
            use_tc_tiling_on_sc=False, needs_layout_passes=False),
        scratch_types=[
            pltpu.VMEM((2, GEKB, EMB), jnp.float32),
            pltpu.VMEM((2, GEKB), jnp.int32),
            pltpu.VMEM((2, GE, NSPH, KPAD), jnp.float32),
            pltpu.VMEM((2, ROWS, _GRP), jnp.float32),
            pltpu.VMEM((1, _L), jnp.int32),
            pltpu.SemaphoreType.DMA,
            pltpu.SemaphoreType.DMA,
            pltpu.SemaphoreType.DMA,
        ],
    )
    def k(m_hbm, ids_hbm, sph_hbm, bnd_hbm, out_hbm, m_v, ids_v, sph_v,
          stg_v, bnd_v, sem_in, sem_out0, sem_out1):
        sem_outs = (sem_out0, sem_out1)
        wid = lax.axis_index("s") * _NC + lax.axis_index("c")
        e_base = wid * EPW
        pltpu.sync_copy(bnd_hbm.at[pl.ds(wid, 1), :], bnd_v)
        ptr0 = bnd_v[0, :][0]
        lane = lax.broadcasted_iota(jnp.int32, (_L,), 0)
        zero = jnp.zeros((_L,), jnp.float32)
        SUBS = _GRP // GE
        NSG = (EPW // _GRP) * SUBS

        def dma_base(ptr):
            d = jnp.minimum(ptr - (ptr % 8), N - GEKB)
            return pl.multiple_of(d, 8)

        def fire(gsg, ptr_est, p):
            sub_e0 = e_base + gsg * GE
            d = dma_base(ptr_est)
            pltpu.async_copy(m_hbm.at[pl.ds(d, GEKB)], m_v.at[p], sem_in)
            pltpu.async_copy(ids_hbm.at[pl.ds(d, GEKB)], ids_v.at[p], sem_in)
            pltpu.async_copy(sph_hbm.at[pl.ds(sub_e0, GE)], sph_v.at[p],
                             sem_in)

        def wait_in(p):
            pltpu.make_async_copy(m_hbm.at[pl.ds(0, GEKB)], m_v.at[p],
                                  sem_in).wait()
            pltpu.make_async_copy(ids_hbm.at[pl.ds(0, GEKB)], ids_v.at[p],
                                  sem_in).wait()
            pltpu.make_async_copy(sph_hbm.at[pl.ds(0, GE)], sph_v.at[p],
                                  sem_in).wait()

        def scan_rows(p, sub_e0):
            def b_body(b, acc):
                idv = ids_v[p, pl.ds(b * _L, _L)]
                hit = jnp.logical_and(idv >= sub_e0, idv < sub_e0 + GE)
                return acc + plsc.all_reduce_population_count(hit)

            cnt = lax.fori_loop(0, GEKB // _L, b_body,
                                jnp.zeros((_L,), jnp.int32))
            return cnt[0]

        def edge_body(p, sp, e_loc, carry):
            ptr, sub_e0, dma_start = carry
            e = sub_e0 + e_loc
            start_local = ptr - dma_start
            base0 = start_local - (start_local % _L)
            base0 = pl.multiple_of(base0, _L)

            def cnt_cond(c):
                base, cnt = c
                may_continue = start_local + cnt >= base
                return jnp.logical_and(may_continue, base + _L <= GEKB)

            def cnt_body(c):
                base, cnt = c
                idv = ids_v[p, pl.ds(base, _L)]
                nm = plsc.all_reduce_population_count(idv == e)[0]
                return (base + _L, cnt + nm)

            _, seg_len = lax.while_loop(cnt_cond, cnt_body, (base0, 0))

            def kb_body(kb, accs):
                k0 = kb * _H
                wvs = []
                for s in range(NSPH):
                    wv = sph_v[p, e_loc, s, pl.ds(k0, _L)]
                    wvs.append(jnp.where(lane + k0 < seg_len, wv, 0.0))
                accs = list(accs)
                for t in range(_H):
                    local = start_local + k0 + t
                    rows = [m_v[p, local, pl.ds(j * _L, _L)]
                            for j in range(NV)]
                    for s in range(NSPH):
                        wsp = lax.broadcast_in_dim(wvs[s][t], (_L,), ())
                        for j in range(NV):
                            accs[s * NV + j] = accs[s * NV + j] + wsp * rows[j]
                return tuple(accs)

            nb = (seg_len + _H - 1) // _H
            accs = lax.fori_loop(0, nb, kb_body, (zero,) * NACC)

            e_col = jnp.broadcast_to((e - e_base) % _GRP, (_L,)).astype(
                jnp.int32)
            sp_idx = jnp.full((_L,), sp, jnp.int32)
            for s in range(NSPH):
                for j in range(NV):
                    idxr = lane + (s * EMB + j * _L)
                    plsc.store_scatter(stg_v, [sp_idx, idxr, e_col],
                                       accs[s * NV + j])
            return (ptr + seg_len, sub_e0, dma_start)

        def run_sub(gsg, ptr, p, sp):
            sub_e0 = e_base + gsg * GE
            dma_start = dma_base(ptr)
            wait_in(p)
            nxt_ptr = ptr + scan_rows(p, sub_e0)

            @pl.when(gsg + 1 < NSG)
            def _():
                fire(gsg + 1, nxt_ptr, 1 - p)

            body = functools.partial(edge_body, p, sp)
            ptr, _, _ = lax.fori_loop(0, GE, body, (ptr, sub_e0, dma_start))
            return ptr

        def pair_body(i, ptr):
            for half in range(2):
                g = 2 * i + half

                @pl.when(i >= 1)
                def _():
                    pltpu.make_async_copy(
                        stg_v.at[half],
                        out_hbm.at[:, pl.ds(0, _GRP)], sem_outs[half]).wait()

                for sub in range(SUBS):
                    j = half * SUBS + sub
                    ptr = run_sub(g * SUBS + sub, ptr, j % 2, half)
                col0 = e_base + g * _GRP
                pltpu.async_copy(stg_v.at[half],
                                 out_hbm.at[:, pl.ds(col0, _GRP)],
                                 sem_outs[half])
            return ptr

        fire(0, ptr0, 0)
        lax.fori_loop(0, EPW // _GRP // 2, pair_body, ptr0)
        for half in range(2):
            pltpu.make_async_copy(stg_v.at[half],
                                  out_hbm.at[:, pl.ds(0, _GRP)],
                                  sem_outs[half]).wait()

    return k(m, ids, sph_p, bounds2d)


def _tc_body(a_ref, st_ref, wf_ref, out_ref):
    INTERM, NSPH, BE = a_ref.shape
    EMB = st_ref.shape[0] // NSPH
    st = st_ref[...].reshape(NSPH, EMB, BE)
    a = a_ref[...]
    h = jnp.zeros((INTERM, EMB, BE), jnp.float32)
    for s in range(NSPH):
        h = h + a[:, s, :][:, None, :] * st[s][None, :, :]
    hf = h.reshape(INTERM * EMB, BE)
    out_ref[...] = lax.dot_general(
        hf, wf_ref[...], (((0,), (0,)), ((), ())),
        preferred_element_type=jnp.float32)


def _tc_compute(a_t, sumk_t, wf, BE=512):
    INTERM, NSPH, E = a_t.shape
    ROWS = sumk_t.shape[0]
    WK, UNITS = wf.shape
    grid = (E // BE,)
    return pl.pallas_call(
        _tc_body,
        grid=grid,
        in_specs=[
            pl.BlockSpec((INTERM, NSPH, BE), lambda i: (0, 0, i)),
            pl.BlockSpec((ROWS, BE), lambda i: (0, i)),
            pl.BlockSpec((WK, UNITS), lambda i: (0, 0)),
        ],
        out_specs=pl.BlockSpec((BE, UNITS), lambda i: (i, 0)),
        out_shape=jax.ShapeDtypeStruct((E, UNITS), jnp.float32),
        compiler_params=pltpu.CompilerParams(
            dimension_semantics=("arbitrary",)),
    )(a_t, sumk_t, wf)


def kernel(rbf_W1, sph, m, weight, id_reduce, id_ragged_idx):
    E, INTERM, NSPH = rbf_W1.shape
    Kmax = sph.shape[2]
    N, EMB = m.shape
    UNITS = weight.shape[2]

    ids = id_reduce.astype(jnp.int32)
    EPW = E // _NW
    qs = jnp.arange(_NW + 1, dtype=jnp.int32) * EPW
    bounds = jnp.searchsorted(ids, qs, side="left").astype(jnp.int32)
    bounds = jnp.concatenate([bounds, jnp.full((15,), N, jnp.int32)])
    bounds2d = jnp.tile(bounds[:, None], (1, _L))

    KPAD = (Kmax + _L - 1) // _L * _L + (_L - _H)
    sph_p = jnp.pad(sph, ((0, 0), (0, 0), (0, KPAD - Kmax)))

    sumk_t = _sc_sumk(m, ids, sph_p, bounds2d, Kmax)
    a_t = jnp.transpose(rbf_W1, (1, 2, 0))
    wf = jnp.transpose(weight, (1, 0, 2)).reshape(INTERM * EMB, UNITS)
    return _tc_compute(a_t, sumk_t, wf)


# speedup vs baseline: 1.2333x; 1.1697x over previous
import functools

import jax
import jax.numpy as jnp
from jax import lax
from jax.experimental import pallas as pl
from jax.experimental.pallas import tpu as pltpu
from jax.experimental.pallas import tpu_sc as plsc

_NC, _NS = 2, 16
_NW = _NC * _NS
_L = 16
_H = 16
_GRP = 16


def _pick_ge(KPAD):
    for ge in (16, 8, 4, 2):
        if ge * KPAD <= 1024 and _GRP % ge == 0:
            return ge
    return 1


def _sc_sumk(m, ids, sph_p, bounds2d, Kmax, E0, EH):
    N, EMB = m.shape
    E, NSPH, KPAD = sph_p.shape
    EPW = EH // _NW
    GE = _pick_ge(Kmax)
    GEKB = (GE * Kmax + 8 + _H + 15) // 16 * 16
    NV = EMB // _L
    NACC = NSPH * NV
    ROWS = NSPH * EMB
    CNTB = GEKB // _L
    mesh = plsc.VectorSubcoreMesh(core_axis_name="c", subcore_axis_name="s")

    @functools.partial(
        pl.kernel,
        mesh=mesh,
        out_type=jax.ShapeDtypeStruct((ROWS, EH), jnp.float32),
        compiler_params=pltpu.CompilerParams(
            use_tc_tiling_on_sc=False, needs_layout_passes=False),
        scratch_types=[
            pltpu.VMEM((2, GEKB, EMB), jnp.float32),
            pltpu.VMEM((2, GEKB), jnp.int32),
            pltpu.VMEM((2, GE, NSPH, KPAD), jnp.float32),
            pltpu.VMEM((2, ROWS, _GRP), jnp.float32),
            pltpu.VMEM((1, _L), jnp.int32),
            pltpu.SemaphoreType.DMA,
            pltpu.SemaphoreType.DMA,
            pltpu.SemaphoreType.DMA,
        ],
    )
    def k(m_hbm, ids_hbm, sph_hbm, bnd_hbm, out_hbm, m_v, ids_v, sph_v,
          stg_v, bnd_v, sem_in, sem_out0, sem_out1):
        sem_outs = (sem_out0, sem_out1)
        wid = lax.axis_index("s") * _NC + lax.axis_index("c")
        le_base = wid * EPW
        e_base = E0 + le_base
        pltpu.sync_copy(bnd_hbm.at[pl.ds(wid, 1), :], bnd_v)
        ptr0 = bnd_v[0, :][0]
        lane = lax.broadcasted_iota(jnp.int32, (_L,), 0)
        zero = jnp.zeros((_L,), jnp.float32)
        SUBS = _GRP // GE
        NSG = (EPW // _GRP) * SUBS

        def dma_base(ptr):
            d = jnp.minimum(ptr - (ptr % 8), N - GEKB)
            return pl.multiple_of(d, 8)

        def fire(gsg, ptr_est, p):
            sub_e0 = e_base + gsg * GE
            d = dma_base(ptr_est)
            pltpu.async_copy(m_hbm.at[pl.ds(d, GEKB)], m_v.at[p], sem_in)
            pltpu.async_copy(ids_hbm.at[pl.ds(d, GEKB)], ids_v.at[p], sem_in)
            pltpu.async_copy(sph_hbm.at[pl.ds(sub_e0, GE)], sph_v.at[p],
                             sem_in)

        def wait_in(p):
            pltpu.make_async_copy(m_hbm.at[pl.ds(0, GEKB)], m_v.at[p],
                                  sem_in).wait()
            pltpu.make_async_copy(ids_hbm.at[pl.ds(0, GEKB)], ids_v.at[p],
                                  sem_in).wait()
            pltpu.make_async_copy(sph_hbm.at[pl.ds(0, GE)], sph_v.at[p],
                                  sem_in).wait()

        def scan_rows(p, sub_e0):
            def b_body(b, acc):
                idv = ids_v[p, pl.ds(b * _L, _L)]
                hit = jnp.logical_and(idv >= sub_e0, idv < sub_e0 + GE)
                return acc + plsc.all_reduce_population_count(hit)

            cnt = lax.fori_loop(0, GEKB // _L, b_body,
                                jnp.zeros((_L,), jnp.int32))
            return cnt[0]

        def edge_body(p, sp, e_loc, carry):
            ptr, sub_e0, dma_start = carry
            e = sub_e0 + e_loc
            start_local = ptr - dma_start
            base0 = start_local - (start_local % _L)
            base0 = pl.multiple_of(base0, _L)

            def cnt_cond(c):
                base, cnt = c
                may_continue = start_local + cnt >= base
                return jnp.logical_and(may_continue, base + _L <= GEKB)

            def cnt_body(c):
                base, cnt = c
                idv = ids_v[p, pl.ds(base, _L)]
                nm = plsc.all_reduce_population_count(idv == e)[0]
                return (base + _L, cnt + nm)

            _, seg_len = lax.while_loop(cnt_cond, cnt_body, (base0, 0))

            def kb_body(kb, accs):
                k0 = kb * _H
                wvs = []
                for s in range(NSPH):
                    wv = sph_v[p, e_loc, s, pl.ds(k0, _L)]
                    wvs.append(jnp.where(lane + k0 < seg_len, wv, 0.0))
                accs = list(accs)
                for t in range(_H):
                    local = start_local + k0 + t
                    rows = [m_v[p, local, pl.ds(j * _L, _L)]
                            for j in range(NV)]
                    for s in range(NSPH):
                        wsp = lax.broadcast_in_dim(wvs[s][t], (_L,), ())
                        for j in range(NV):
                            accs[s * NV + j] = accs[s * NV + j] + wsp * rows[j]
                return tuple(accs)

            nb = (seg_len + _H - 1) // _H
            accs = lax.fori_loop(0, nb, kb_body, (zero,) * NACC)

            e_col = jnp.broadcast_to((e - e_base) % _GRP, (_L,)).astype(
                jnp.int32)
            sp_idx = jnp.full((_L,), sp, jnp.int32)
            for s in range(NSPH):
                for j in range(NV):
                    idxr = lane + (s * EMB + j * _L)
                    plsc.store_scatter(stg_v, [sp_idx, idxr, e_col],
                                       accs[s * NV + j])
            return (ptr + seg_len, sub_e0, dma_start)

        def run_sub(gsg, ptr, p, sp):
            sub_e0 = e_base + gsg * GE
            dma_start = dma_base(ptr)
            wait_in(p)
            nxt_ptr = ptr + scan_rows(p, sub_e0)

            @pl.when(gsg + 1 < NSG)
            def _():
                fire(gsg + 1, nxt_ptr, 1 - p)

            body = functools.partial(edge_body, p, sp)
            ptr, _, _ = lax.fori_loop(0, GE, body, (ptr, sub_e0, dma_start))
            return ptr

        def pair_body(i, ptr):
            for half in range(2):
                g = 2 * i + half

                @pl.when(i >= 1)
                def _():
                    pltpu.make_async_copy(
                        stg_v.at[half],
                        out_hbm.at[:, pl.ds(0, _GRP)], sem_outs[half]).wait()

                for sub in range(SUBS):
                    j = half * SUBS + sub
                    ptr = run_sub(g * SUBS + sub, ptr, j % 2, half)
                col0 = le_base + g * _GRP
                pltpu.async_copy(stg_v.at[half],
                                 out_hbm.at[:, pl.ds(col0, _GRP)],
                                 sem_outs[half])
            return ptr

        fire(0, ptr0, 0)
        lax.fori_loop(0, EPW // _GRP // 2, pair_body, ptr0)
        for half in range(2):
            pltpu.make_async_copy(stg_v.at[half],
                                  out_hbm.at[:, pl.ds(0, _GRP)],
                                  sem_outs[half]).wait()

    return k(m, ids, sph_p, bounds2d)


def _tc_body(a_ref, st_ref, wf_ref, out_ref):
    INTERM, NSPH, BE = a_ref.shape
    EMB = st_ref.shape[0] // NSPH
    st = st_ref[...].reshape(NSPH, EMB, BE)
    a = a_ref[...]
    h = jnp.zeros((INTERM, EMB, BE), jnp.float32)
    for s in range(NSPH):
        h = h + a[:, s, :][:, None, :] * st[s][None, :, :]
    hf = h.reshape(INTERM * EMB, BE)
    out_ref[...] = lax.dot_general(
        hf, wf_ref[...], (((0,), (0,)), ((), ())),
        preferred_element_type=jnp.float32)


def _tc_compute(a_t, sumk_t, wf, E0, BE=512):
    INTERM, NSPH, E = a_t.shape
    ROWS, EH = sumk_t.shape
    WK, UNITS = wf.shape
    boff = E0 // BE
    grid = (EH // BE,)
    return pl.pallas_call(
        _tc_body,
        grid=grid,
        in_specs=[
            pl.BlockSpec((INTERM, NSPH, BE), lambda i: (0, 0, i + boff)),
            pl.BlockSpec((ROWS, BE), lambda i: (0, i)),
            pl.BlockSpec((WK, UNITS), lambda i: (0, 0)),
        ],
        out_specs=pl.BlockSpec((BE, UNITS), lambda i: (i, 0)),
        out_shape=jax.ShapeDtypeStruct((EH, UNITS), jnp.float32),
        compiler_params=pltpu.CompilerParams(
            dimension_semantics=("arbitrary",)),
    )(a_t, sumk_t, wf)


def kernel(rbf_W1, sph, m, weight, id_reduce, id_ragged_idx):
    E, INTERM, NSPH = rbf_W1.shape
    Kmax = sph.shape[2]
    N, EMB = m.shape
    UNITS = weight.shape[2]

    ids = id_reduce.astype(jnp.int32)

    KPAD = (Kmax + _L - 1) // _L * _L + (_L - _H)
    sph_p = jnp.pad(sph, ((0, 0), (0, 0), (0, KPAD - Kmax)))
    a_t = jnp.transpose(rbf_W1, (1, 2, 0))
    wf = jnp.transpose(weight, (1, 0, 2)).reshape(INTERM * EMB, UNITS)

    HALVES = 2
    EH = E // HALVES
    EPW = EH // _NW
    outs = []
    for h in range(HALVES):
        E0 = h * EH
        qs = E0 + jnp.arange(_NW + 1, dtype=jnp.int32) * EPW
        bounds = jnp.searchsorted(ids, qs, side="left").astype(jnp.int32)
        bounds = jnp.concatenate([bounds, jnp.full((15,), N, jnp.int32)])
        bounds2d = jnp.tile(bounds[:, None], (1, _L))
        sumk_t = _sc_sumk(m, ids, sph_p, bounds2d, Kmax, E0, EH)
        outs.append(_tc_compute(a_t, sumk_t, wf, E0))
    return jnp.concatenate(outs, axis=0)


# speedup vs baseline: 1.3045x; 1.0577x over previous
import functools

import jax
import jax.numpy as jnp
from jax import lax
from jax.experimental import pallas as pl
from jax.experimental.pallas import tpu as pltpu
from jax.experimental.pallas import tpu_sc as plsc

_NC, _NS = 2, 16
_NW = _NC * _NS
_L = 16
_H = 16
_GRP = 16


def _pick_ge(KPAD):
    for ge in (16, 8, 4, 2):
        if ge * KPAD <= 1024 and _GRP % ge == 0:
            return ge
    return 1


def _sc_sumk(m, ids, sph_p, bounds2d, Kmax, E0, EH):
    N, EMB = m.shape
    E, NSPH, KPAD = sph_p.shape
    EPW = EH // _NW
    GE = _pick_ge(Kmax)
    GEKB = (GE * Kmax + 8 + _H + 15) // 16 * 16
    NV = EMB // _L
    NACC = NSPH * NV
    ROWS = NSPH * EMB
    CNTB = GEKB // _L
    mesh = plsc.VectorSubcoreMesh(core_axis_name="c", subcore_axis_name="s")

    @functools.partial(
        pl.kernel,
        mesh=mesh,
        out_type=jax.ShapeDtypeStruct((ROWS, EH), jnp.float32),
        compiler_params=pltpu.CompilerParams(
            use_tc_tiling_on_sc=False, needs_layout_passes=False),
        scratch_types=[
            pltpu.VMEM((2, GEKB, EMB), jnp.float32),
            pltpu.VMEM((2, GEKB), jnp.int32),
            pltpu.VMEM((2, GE, NSPH, KPAD), jnp.float32),
            pltpu.VMEM((2, ROWS, _GRP), jnp.float32),
            pltpu.VMEM((1, _L), jnp.int32),
            pltpu.SemaphoreType.DMA,
            pltpu.SemaphoreType.DMA,
            pltpu.SemaphoreType.DMA,
        ],
    )
    def k(m_hbm, ids_hbm, sph_hbm, bnd_hbm, out_hbm, m_v, ids_v, sph_v,
          stg_v, bnd_v, sem_in, sem_out0, sem_out1):
        sem_outs = (sem_out0, sem_out1)
        wid = lax.axis_index("s") * _NC + lax.axis_index("c")
        le_base = wid * EPW
        e_base = E0 + le_base
        pltpu.sync_copy(bnd_hbm.at[pl.ds(wid, 1), :], bnd_v)
        ptr0 = bnd_v[0, :][0]
        lane = lax.broadcasted_iota(jnp.int32, (_L,), 0)
        zero = jnp.zeros((_L,), jnp.float32)
        SUBS = _GRP // GE
        NSG = (EPW // _GRP) * SUBS

        def dma_base(ptr):
            d = jnp.minimum(ptr - (ptr % 8), N - GEKB)
            return pl.multiple_of(d, 8)

        def fire(gsg, ptr_est, p):
            sub_e0 = e_base + gsg * GE
            d = dma_base(ptr_est)
            pltpu.async_copy(m_hbm.at[pl.ds(d, GEKB)], m_v.at[p], sem_in)
            pltpu.async_copy(ids_hbm.at[pl.ds(d, GEKB)], ids_v.at[p], sem_in)
            pltpu.async_copy(sph_hbm.at[pl.ds(sub_e0, GE)], sph_v.at[p],
                             sem_in)

        def wait_in(p):
            pltpu.make_async_copy(m_hbm.at[pl.ds(0, GEKB)], m_v.at[p],
                                  sem_in).wait()
            pltpu.make_async_copy(ids_hbm.at[pl.ds(0, GEKB)], ids_v.at[p],
                                  sem_in).wait()
            pltpu.make_async_copy(sph_hbm.at[pl.ds(0, GE)], sph_v.at[p],
                                  sem_in).wait()

        def scan_rows(p, sub_e0):
            def b_body(b, acc):
                idv = ids_v[p, pl.ds(b * _L, _L)]
                hit = jnp.logical_and(idv >= sub_e0, idv < sub_e0 + GE)
                return acc + plsc.all_reduce_population_count(hit)

            cnt = lax.fori_loop(0, GEKB // _L, b_body,
                                jnp.zeros((_L,), jnp.int32))
            return cnt[0]

        def edge_body(p, sp, e_loc, carry):
            ptr, sub_e0, dma_start = carry
            e = sub_e0 + e_loc
            start_local = ptr - dma_start
            base0 = start_local - (start_local % _L)
            base0 = pl.multiple_of(base0, _L)

            def cnt_cond(c):
                base, cnt = c
                may_continue = start_local + cnt >= base
                return jnp.logical_and(may_continue, base + _L <= GEKB)

            def cnt_body(c):
                base, cnt = c
                idv = ids_v[p, pl.ds(base, _L)]
                nm = plsc.all_reduce_population_count(idv == e)[0]
                return (base + _L, cnt + nm)

            _, seg_len = lax.while_loop(cnt_cond, cnt_body, (base0, 0))

            def kb_body(kb, accs):
                k0 = kb * _H
                wvs = []
                for s in range(NSPH):
                    wv = sph_v[p, e_loc, s, pl.ds(k0, _L)]
                    wvs.append(jnp.where(lane + k0 < seg_len, wv, 0.0))
                accs = list(accs)
                for t in range(_H):
                    local = start_local + k0 + t
                    rows = [m_v[p, local, pl.ds(j * _L, _L)]
                            for j in range(NV)]
                    for s in range(NSPH):
                        wsp = lax.broadcast_in_dim(wvs[s][t], (_L,), ())
                        for j in range(NV):
                            accs[s * NV + j] = accs[s * NV + j] + wsp * rows[j]
                return tuple(accs)

            nb = (seg_len + _H - 1) // _H
            accs = lax.fori_loop(0, nb, kb_body, (zero,) * NACC)

            e_col = jnp.broadcast_to((e - e_base) % _GRP, (_L,)).astype(
                jnp.int32)
            sp_idx = jnp.full((_L,), sp, jnp.int32)
            for s in range(NSPH):
                for j in range(NV):
                    idxr = lane + (s * EMB + j * _L)
                    plsc.store_scatter(stg_v, [sp_idx, idxr, e_col],
                                       accs[s * NV + j])
            return (ptr + seg_len, sub_e0, dma_start)

        def run_sub(gsg, ptr, p, sp):
            sub_e0 = e_base + gsg * GE
            dma_start = dma_base(ptr)
            wait_in(p)
            nxt_ptr = ptr + scan_rows(p, sub_e0)

            @pl.when(gsg + 1 < NSG)
            def _():
                fire(gsg + 1, nxt_ptr, 1 - p)

            body = functools.partial(edge_body, p, sp)
            ptr, _, _ = lax.fori_loop(0, GE, body, (ptr, sub_e0, dma_start))
            return ptr

        def pair_body(i, ptr):
            for half in range(2):
                g = 2 * i + half

                @pl.when(i >= 1)
                def _():
                    pltpu.make_async_copy(
                        stg_v.at[half],
                        out_hbm.at[:, pl.ds(0, _GRP)], sem_outs[half]).wait()

                for sub in range(SUBS):
                    j = half * SUBS + sub
                    ptr = run_sub(g * SUBS + sub, ptr, j % 2, half)
                col0 = le_base + g * _GRP
                pltpu.async_copy(stg_v.at[half],
                                 out_hbm.at[:, pl.ds(col0, _GRP)],
                                 sem_outs[half])
            return ptr

        fire(0, ptr0, 0)
        lax.fori_loop(0, EPW // _GRP // 2, pair_body, ptr0)
        for half in range(2):
            pltpu.make_async_copy(stg_v.at[half],
                                  out_hbm.at[:, pl.ds(0, _GRP)],
                                  sem_outs[half]).wait()

    return k(m, ids, sph_p, bounds2d)


def _tc_body(a_ref, st_ref, wf_ref, out_ref):
    INTERM, NSPH, BE = a_ref.shape
    EMB = st_ref.shape[0] // NSPH
    st = st_ref[...].reshape(NSPH, EMB, BE)
    a = a_ref[...]
    h = jnp.zeros((INTERM, EMB, BE), jnp.float32)
    for s in range(NSPH):
        h = h + a[:, s, :][:, None, :] * st[s][None, :, :]
    hf = h.reshape(INTERM * EMB, BE)
    out_ref[...] = lax.dot_general(
        hf, wf_ref[...], (((0,), (0,)), ((), ())),
        preferred_element_type=jnp.float32)


def _tc_compute(a_t, sumk_t, wf, E0, BE=512):
    INTERM, NSPH, E = a_t.shape
    ROWS, EH = sumk_t.shape
    WK, UNITS = wf.shape
    boff = E0 // BE
    grid = (EH // BE,)
    return pl.pallas_call(
        _tc_body,
        grid=grid,
        in_specs=[
            pl.BlockSpec((INTERM, NSPH, BE), lambda i: (0, 0, i + boff)),
            pl.BlockSpec((ROWS, BE), lambda i: (0, i)),
            pl.BlockSpec((WK, UNITS), lambda i: (0, 0)),
        ],
        out_specs=pl.BlockSpec((BE, UNITS), lambda i: (i, 0)),
        out_shape=jax.ShapeDtypeStruct((EH, UNITS), jnp.float32),
        compiler_params=pltpu.CompilerParams(
            dimension_semantics=("arbitrary",)),
    )(a_t, sumk_t, wf)


def kernel(rbf_W1, sph, m, weight, id_reduce, id_ragged_idx):
    E, INTERM, NSPH = rbf_W1.shape
    Kmax = sph.shape[2]
    N, EMB = m.shape
    UNITS = weight.shape[2]

    ids = id_reduce.astype(jnp.int32)

    KPAD = (Kmax + _L - 1) // _L * _L + (_L - _H)
    sph_p = jnp.pad(sph, ((0, 0), (0, 0), (0, KPAD - Kmax)))
    a_t = jnp.transpose(rbf_W1, (1, 2, 0))
    wf = jnp.transpose(weight, (1, 0, 2)).reshape(INTERM * EMB, UNITS)

    HALVES = 4
    EH = E // HALVES
    EPW = EH // _NW
    outs = []
    for h in range(HALVES):
        E0 = h * EH
        qs = E0 + jnp.arange(_NW + 1, dtype=jnp.int32) * EPW
        bounds = jnp.searchsorted(ids, qs, side="left").astype(jnp.int32)
        bounds = jnp.concatenate([bounds, jnp.full((15,), N, jnp.int32)])
        bounds2d = jnp.tile(bounds[:, None], (1, _L))
        sumk_t = _sc_sumk(m, ids, sph_p, bounds2d, Kmax, E0, EH)
        outs.append(_tc_compute(a_t, sumk_t, wf, E0))
    return jnp.concatenate(outs, axis=0)
